# baseline (speedup 1.0000x reference)
import functools

import jax
import jax.numpy as jnp
from jax import lax
from jax.experimental import pallas as pl
from jax.experimental.pallas import tpu as pltpu
from jax.experimental.pallas import tpu_sc as plsc

N_NODES = 10000
N_PAD = 10112
D = 128
E = 320000
NC = 2
NS = 16
NW = NC * NS
CHUNK = 128
N_CHUNKS = E // CHUNK
CPW = 80
NPASS = 2
CPP = CPW // NPASS
LAST_CNT = N_CHUNKS - (NW - 1) * CPW
ROWS_PER_SUB = N_PAD // NS

_SC_MESH = plsc.VectorSubcoreMesh(core_axis_name="c", subcore_axis_name="s")


@functools.partial(
    pl.kernel,
    out_type=jax.ShapeDtypeStruct((NC * N_PAD, D), jnp.float32),
    mesh=_SC_MESH,
    scratch_types=[
        pltpu.VMEM_SHARED((N_PAD, D), jnp.float32),
        pltpu.VMEM((CPP, CHUNK), jnp.int32),
        pltpu.VMEM((CPP, CHUNK), jnp.int32),
        pltpu.VMEM((CHUNK, D), jnp.float32),
        pltpu.VMEM((CHUNK, D), jnp.float32),
        pltpu.SemaphoreType.DMA,
        pltpu.SemaphoreType.DMA,
        pltpu.SemaphoreType.DMA,
        pltpu.SemaphoreType.DMA,
    ],
)
def _sc_aggregate(x_hbm, src_hbm, dst_hbm, srct_hbm, dstt_hbm, out_hbm,
                  aggr_sh, src_v, dst_v, rows0, rows1, sg0, sg1, ss0, ss1):
    c = lax.axis_index("c")
    s = lax.axis_index("s")
    w = c * NS + s

    r0 = s * ROWS_PER_SUB
    rows = (rows0, rows1)
    sg = (sg0, sg1)
    ss = (ss0, ss1)

    last = w == NW - 1
    count = jnp.where(last, LAST_CNT, CPP)
    for p in range(NPASS):
        base = w * CPW + p * CPP
        if p == 0:
            @pl.when(jnp.logical_not(last))
            def _():
                pltpu.sync_copy(src_hbm.at[pl.ds(base, CPP)], src_v)
                pltpu.sync_copy(dst_hbm.at[pl.ds(base, CPP)], dst_v)

            @pl.when(last)
            def _():
                t0 = LAST_CNT - 4
                pltpu.sync_copy(src_hbm.at[pl.ds(base, t0)],
                                src_v.at[pl.ds(0, t0)])
                pltpu.sync_copy(dst_hbm.at[pl.ds(base, t0)],
                                dst_v.at[pl.ds(0, t0)])
                for k in range(4):
                    pltpu.sync_copy(srct_hbm.at[pl.ds(k * CHUNK, CHUNK)],
                                    src_v.at[t0 + k])
                    pltpu.sync_copy(dstt_hbm.at[pl.ds(k * CHUNK, CHUNK)],
                                    dst_v.at[t0 + k])
        else:
            @pl.when(jnp.logical_not(last))
            def _():
                pltpu.sync_copy(src_hbm.at[pl.ds(base, CPP)], src_v)
                pltpu.sync_copy(dst_hbm.at[pl.ds(base, CPP)], dst_v)

        pcount = count if p == 0 else jnp.where(last, 0, CPP)

        @pl.when(pcount > 0)
        def _():
            pltpu.async_copy(x_hbm.at[src_v.at[0]], rows0, sg0)
            pltpu.async_copy(x_hbm.at[src_v.at[1]], rows1, sg1)

        if p == 0:
            @pl.when(s < NS - 1)
            def _():
                pltpu.sync_copy(x_hbm.at[pl.ds(r0, ROWS_PER_SUB)],
                                aggr_sh.at[pl.ds(r0, ROWS_PER_SUB)])

            @pl.when(s == NS - 1)
            def _():
                pltpu.sync_copy(
                    x_hbm.at[pl.ds(r0, N_NODES - (NS - 1) * ROWS_PER_SUB)],
                    aggr_sh.at[pl.ds(r0, N_NODES - (NS - 1) * ROWS_PER_SUB)])

            plsc.subcore_barrier()

        def pair(g, carry):
            for b in (0, 1):
                j = g * 2 + b

                @pl.when(j < pcount)
                def _(b=b, j=j):
                    pltpu.make_async_copy(x_hbm.at[src_v.at[j]], rows[b],
                                          sg[b]).wait()
                    pltpu.async_copy(rows[b], aggr_sh.at[dst_v.at[j]], ss[b],
                                     add=True)
                    pltpu.make_async_copy(rows[b], aggr_sh.at[dst_v.at[j]],
                                          ss[b]).wait()
                    @pl.when(j + 2 < pcount)
                    def _():
                        pltpu.async_copy(x_hbm.at[src_v.at[j + 2]], rows[b],
                                         sg[b])
            return carry

        lax.fori_loop(0, CPP // 2, pair, 0)

    plsc.subcore_barrier()
    pltpu.sync_copy(aggr_sh.at[pl.ds(r0, ROWS_PER_SUB)],
                    out_hbm.at[pl.ds(c * N_PAD + r0, ROWS_PER_SUB)])


def _tc_block(x_ref, p_ref, w1_ref, b1_ref, w2_ref, b2_ref, g_ref, bt_ref,
              wr_ref, br_ref, o_ref):
    x = x_ref[...]
    h = p_ref[0] + p_ref[1] - x
    cdims = (((1,), (1,)), ((), ()))
    a = lax.dot_general(h, w1_ref[...], cdims,
                        preferred_element_type=jnp.float32) + b1_ref[...]
    a = jnp.maximum(a, 0.0)
    hh = lax.dot_general(a, w2_ref[...], cdims,
                         preferred_element_type=jnp.float32) + b2_ref[...]
    mean = jnp.mean(hh, axis=1, keepdims=True)
    cen = hh - mean
    var = jnp.mean(cen * cen, axis=1, keepdims=True)
    hn = cen * lax.rsqrt(var + 1e-5) * g_ref[...] + bt_ref[...]
    res = lax.dot_general(x, wr_ref[...], cdims,
                          preferred_element_type=jnp.float32) + br_ref[...]
    o_ref[...] = hn + res


_ROWS_BLK = 10000


def kernel(x, edge_index, W1, b1, W2, b2, gamma, beta, Wres, bres):
    x = x.astype(jnp.float32)
    ei = edge_index.astype(jnp.int32)
    src = ei[0]
    dst = ei[1]

    src_p = src.reshape(N_CHUNKS, CHUNK)
    dst_p = dst.reshape(N_CHUNKS, CHUNK)

    tail = (N_CHUNKS - 4) * CHUNK
    partials = _sc_aggregate(x, src_p, dst_p, src[tail:], dst[tail:]).reshape(
        NC, N_PAD, D)

    n_blk = N_NODES // _ROWS_BLK
    full = pl.BlockSpec((128, 128), lambda i: (0, 0))
    vec = pl.BlockSpec((1, 128), lambda i: (0, 0))
    out = pl.pallas_call(
        _tc_block,
        grid=(n_blk,),
        in_specs=[
            pl.BlockSpec((_ROWS_BLK, D), lambda i: (i, 0)),
            pl.BlockSpec((NC, _ROWS_BLK, D), lambda i: (0, i, 0)),
            full, vec, full, vec, vec, vec, full, vec,
        ],
        out_specs=pl.BlockSpec((_ROWS_BLK, D), lambda i: (i, 0)),
        out_shape=jax.ShapeDtypeStruct((N_NODES, D), jnp.float32),
    )(x, partials, W1, b1.reshape(1, D), W2, b2.reshape(1, D),
      gamma.reshape(1, D), beta.reshape(1, D), Wres, bres.reshape(1, D))
    return out


# speedup vs baseline: 1.0146x; 1.0146x over previous
import functools

import jax
import jax.numpy as jnp
from jax import lax
from jax.experimental import pallas as pl
from jax.experimental.pallas import tpu as pltpu
from jax.experimental.pallas import tpu_sc as plsc

N_NODES = 10000
N_PAD = 10112
D = 128
E = 320000
NC = 2
NS = 16
NW = NC * NS
CHUNK = 128
N_CHUNKS = E // CHUNK
CPW = 80
NPASS = 2
CPP = CPW // NPASS
LAST_CNT = N_CHUNKS - (NW - 1) * CPW
ROWS_PER_SUB = N_PAD // NS

_SC_MESH = plsc.VectorSubcoreMesh(core_axis_name="c", subcore_axis_name="s")


@functools.partial(
    pl.kernel,
    out_type=jax.ShapeDtypeStruct((NC * N_PAD, D), jnp.float32),
    mesh=_SC_MESH,
    scratch_types=[
        pltpu.VMEM_SHARED((N_PAD, D), jnp.float32),
        pltpu.VMEM((CPP, CHUNK), jnp.int32),
        pltpu.VMEM((CPP, CHUNK), jnp.int32),
        pltpu.VMEM((CHUNK, D), jnp.float32),
        pltpu.VMEM((CHUNK, D), jnp.float32),
        pltpu.SemaphoreType.DMA,
        pltpu.SemaphoreType.DMA,
        pltpu.SemaphoreType.DMA,
        pltpu.SemaphoreType.DMA,
    ],
)
def _sc_aggregate(x_hbm, src_hbm, dst_hbm, srct_hbm, dstt_hbm, out_hbm,
                  aggr_sh, src_v, dst_v, rows0, rows1, sg0, sg1, ss0, ss1):
    c = lax.axis_index("c")
    s = lax.axis_index("s")
    w = c * NS + s

    r0 = s * ROWS_PER_SUB
    rows = (rows0, rows1)
    sg = (sg0, sg1)
    ss = (ss0, ss1)

    last = w == NW - 1
    count = jnp.where(last, LAST_CNT, CPP)
    for p in range(NPASS):
        base = w * CPW + p * CPP
        if p == 0:
            @pl.when(jnp.logical_not(last))
            def _():
                pltpu.sync_copy(src_hbm.at[pl.ds(base, CPP)], src_v)
                pltpu.sync_copy(dst_hbm.at[pl.ds(base, CPP)], dst_v)

            @pl.when(last)
            def _():
                t0 = LAST_CNT - 4
                pltpu.sync_copy(src_hbm.at[pl.ds(base, t0)],
                                src_v.at[pl.ds(0, t0)])
                pltpu.sync_copy(dst_hbm.at[pl.ds(base, t0)],
                                dst_v.at[pl.ds(0, t0)])
                for k in range(4):
                    pltpu.sync_copy(srct_hbm.at[pl.ds(k * CHUNK, CHUNK)],
                                    src_v.at[t0 + k])
                    pltpu.sync_copy(dstt_hbm.at[pl.ds(k * CHUNK, CHUNK)],
                                    dst_v.at[t0 + k])
        else:
            @pl.when(jnp.logical_not(last))
            def _():
                pltpu.sync_copy(src_hbm.at[pl.ds(base, CPP)], src_v)
                pltpu.sync_copy(dst_hbm.at[pl.ds(base, CPP)], dst_v)

        pcount = count if p == 0 else jnp.where(last, 0, CPP)

        @pl.when(pcount > 0)
        def _():
            pltpu.async_copy(x_hbm.at[src_v.at[0]], rows0, sg0)
            pltpu.async_copy(x_hbm.at[src_v.at[1]], rows1, sg1)

        if p == 0:
            @pl.when(s < NS - 1)
            def _():
                pltpu.sync_copy(x_hbm.at[pl.ds(r0, ROWS_PER_SUB)],
                                aggr_sh.at[pl.ds(r0, ROWS_PER_SUB)])

            @pl.when(s == NS - 1)
            def _():
                pltpu.sync_copy(
                    x_hbm.at[pl.ds(r0, N_NODES - (NS - 1) * ROWS_PER_SUB)],
                    aggr_sh.at[pl.ds(r0, N_NODES - (NS - 1) * ROWS_PER_SUB)])

            plsc.subcore_barrier()

        def pair(g, carry):
            for b in (0, 1):
                j = g * 2 + b

                @pl.when(j < pcount)
                def _(b=b, j=j):
                    pltpu.make_async_copy(x_hbm.at[src_v.at[j]], rows[b],
                                          sg[b]).wait()
                    pltpu.async_copy(rows[b], aggr_sh.at[dst_v.at[j]], ss[b],
                                     add=True)
                    pltpu.make_async_copy(rows[b], aggr_sh.at[dst_v.at[j]],
                                          ss[b]).wait()
                    @pl.when(j + 2 < pcount)
                    def _():
                        pltpu.async_copy(x_hbm.at[src_v.at[j + 2]], rows[b],
                                         sg[b])
            return carry

        lax.fori_loop(0, CPP // 2, pair, 0)

    plsc.subcore_barrier()
    pltpu.sync_copy(aggr_sh.at[pl.ds(r0, ROWS_PER_SUB)],
                    out_hbm.at[pl.ds(c * N_PAD + r0, ROWS_PER_SUB)])


def _tc_block(x_ref, p_ref, w1_ref, b1_ref, w2_ref, b2_ref, g_ref, bt_ref,
              wr_ref, br_ref, o_ref):
    x = x_ref[...]
    h = p_ref[0] + p_ref[1] - x
    cdims = (((1,), (1,)), ((), ()))
    a = lax.dot_general(h, w1_ref[...], cdims,
                        preferred_element_type=jnp.float32) + b1_ref[...]
    a = jnp.maximum(a, 0.0)
    hh = lax.dot_general(a, w2_ref[...], cdims,
                         preferred_element_type=jnp.float32) + b2_ref[...]
    mean = jnp.mean(hh, axis=1, keepdims=True)
    cen = hh - mean
    var = jnp.mean(cen * cen, axis=1, keepdims=True)
    hn = cen * lax.rsqrt(var + 1e-5) * g_ref[...] + bt_ref[...]
    res = lax.dot_general(x, wr_ref[...], cdims,
                          preferred_element_type=jnp.float32) + br_ref[...]
    o_ref[...] = hn + res


_ROWS_BLK = 5000


def kernel(x, edge_index, W1, b1, W2, b2, gamma, beta, Wres, bres):
    x = x.astype(jnp.float32)
    ei = edge_index.astype(jnp.int32)
    src = ei[0]
    dst = ei[1]

    src_p = src.reshape(N_CHUNKS, CHUNK)
    dst_p = dst.reshape(N_CHUNKS, CHUNK)

    tail = (N_CHUNKS - 4) * CHUNK
    partials = _sc_aggregate(x, src_p, dst_p, src[tail:], dst[tail:]).reshape(
        NC, N_PAD, D)

    n_blk = N_NODES // _ROWS_BLK
    full = pl.BlockSpec((128, 128), lambda i: (0, 0))
    vec = pl.BlockSpec((1, 128), lambda i: (0, 0))
    out = pl.pallas_call(
        _tc_block,
        grid=(n_blk,),
        in_specs=[
            pl.BlockSpec((_ROWS_BLK, D), lambda i: (i, 0)),
            pl.BlockSpec((NC, _ROWS_BLK, D), lambda i: (0, i, 0)),
            full, vec, full, vec, vec, vec, full, vec,
        ],
        out_specs=pl.BlockSpec((_ROWS_BLK, D), lambda i: (i, 0)),
        out_shape=jax.ShapeDtypeStruct((N_NODES, D), jnp.float32),
    )(x, partials, W1, b1.reshape(1, D), W2, b2.reshape(1, D),
      gamma.reshape(1, D), beta.reshape(1, D), Wres, bres.reshape(1, D))
    return out
